# baseline (speedup 1.0000x reference)
import functools

import jax
import jax.numpy as jnp
from jax import lax
from jax.experimental import pallas as pl
from jax.experimental.pallas import tpu as pltpu
from jax.experimental.pallas import tpu_sc as plsc

_LEVELS = 8
_BETA = 0.25
_CODE_DIMS = 4


def _make_quant_body(nblk, n_rows, n_cols):
    inv = (1.0 + _BETA) / float(n_rows * n_cols)

    nlev = float(_LEVELS - 1)

    def body(z_ref, g_ref, zq_ref, loss_ref, acc_ref):
        z = z_ref[...]
        g0 = g_ref[0, 0]
        step = (g_ref[0, _LEVELS - 1] - g0) * (1.0 / (_LEVELS - 1))
        r = 1.0 / (jnp.exp(z + z) + 1.0)
        x = 1.0 - (r + r)
        best_f = jnp.floor((nlev + 0.5) - nlev * r)
        zq = g0 + best_f * step
        zq_ref[...] = x + (zq - x)
        diff = zq - x
        i = pl.program_id(0)

        @pl.when(i == 0)
        def _init():
            acc_ref[...] = jnp.zeros(acc_ref.shape, jnp.float32)

        acc_ref[...] += jnp.sum(diff * diff, axis=0, keepdims=True)

        @pl.when(i == nblk - 1)
        def _fin():
            loss_ref[...] = jnp.sum(acc_ref[...])[None, None] * inv

    return body


def _quantize(z_e, grid, nblk=4, interpret=False):
    n, d = z_e.shape
    k = grid.shape[0]
    r = n // nblk
    g2 = grid.reshape(1, k)
    return pl.pallas_call(
        _make_quant_body(nblk, n, d),
        grid=(nblk,),
        in_specs=[
            pl.BlockSpec((r, d), lambda i: (i, 0)),
            pl.BlockSpec((1, k), lambda i: (0, 0)),
        ],
        out_specs=[
            pl.BlockSpec((r, d), lambda i: (i, 0)),
            pl.BlockSpec((1, 1), lambda i: (0, 0)),
        ],
        out_shape=[
            jax.ShapeDtypeStruct((n, d), jnp.float32),
            jax.ShapeDtypeStruct((1, 1), jnp.float32),
        ],
        scratch_shapes=[pltpu.VMEM((1, d), jnp.float32)],
        interpret=interpret,
    )(z_e, g2)


def _sc_mixed_perp(zc, n):
    ns = 16
    rpt_n = n // ns
    mrows = rpt_n // 128
    codes = _LEVELS ** _CODE_DIMS
    cpt = codes // ns
    mesh = plsc.VectorSubcoreMesh(
        core_axis_name="c", subcore_axis_name="s", num_cores=1)

    @functools.partial(
        pl.kernel,
        out_type=[
            jax.ShapeDtypeStruct((n // 128, 128), jnp.int32),
            jax.ShapeDtypeStruct((16,), jnp.float32),
        ],
        mesh=mesh,
        compiler_params=pltpu.CompilerParams(use_tc_tiling_on_sc=False),
        scratch_types=[
            pltpu.VMEM((rpt_n,), jnp.float32),
            pltpu.VMEM((rpt_n,), jnp.float32),
            pltpu.VMEM((rpt_n,), jnp.float32),
            pltpu.VMEM((rpt_n,), jnp.float32),
            pltpu.VMEM((mrows, 128), jnp.int32),
            pltpu.VMEM((128,), jnp.int32),
            pltpu.VMEM((cpt,), jnp.int32),
            pltpu.VMEM((16,), jnp.int32),
            pltpu.VMEM((ns * 16,), jnp.int32),
            pltpu.VMEM((16,), jnp.float32),
            pltpu.VMEM_SHARED((codes,), jnp.int32),
            pltpu.VMEM_SHARED((ns * 16,), jnp.int32),
        ],
    )
    def sc_k(zc_hbm, mixed_hbm, perp_hbm, cb0, cb1, cb2, cb3,
             m2_v, ones_v, slice_v, cnt_v, cnt_all_v, out_v,
             counts_sh, cnt_sh):
        sid = lax.axis_index("s")
        base = sid * rpt_n
        pltpu.sync_copy(zc_hbm.at[0, pl.ds(base, rpt_n)], cb0)
        pltpu.sync_copy(zc_hbm.at[1, pl.ds(base, rpt_n)], cb1)
        pltpu.sync_copy(zc_hbm.at[2, pl.ds(base, rpt_n)], cb2)
        pltpu.sync_copy(zc_hbm.at[3, pl.ds(base, rpt_n)], cb3)
        one16 = jnp.ones((16,), jnp.int32)
        zero16 = jnp.zeros((16,), jnp.int32)
        for j in range(128 // 16):
            ones_v[pl.ds(j * 16, 16)] = one16
        for j in range(cpt // 16):
            slice_v[pl.ds(j * 16, 16)] = zero16
        pltpu.sync_copy(slice_v, counts_sh.at[pl.ds(sid * cpt, cpt)])
        cbs = (cb0, cb1, cb2, cb3)

        nlev = float(_LEVELS - 1)

        def quant_row(rr, _):
            for gg in range(8):
                off = (rr * 8 + gg) * 16
                m = jnp.zeros((16,), jnp.int32)
                for k in range(_CODE_DIMS):
                    v = cbs[k][pl.ds(off, 16)]
                    t = jnp.exp(v + v)
                    idx_k = (nlev + 0.5) - nlev / (t + 1.0)
                    m = m + idx_k.astype(jnp.int32) * (_LEVELS ** k)
                m2_v[rr, pl.ds(gg * 16, 16)] = m
            return 0

        lax.fori_loop(0, mrows, quant_row, 0)
        pltpu.sync_copy(m2_v, mixed_hbm.at[pl.ds(sid * mrows, mrows)])
        plsc.subcore_barrier()
        for j in range(mrows):
            pltpu.sync_copy(ones_v, counts_sh.at[m2_v.at[j]], add=True)
        plsc.subcore_barrier()
        pltpu.sync_copy(counts_sh.at[pl.ds(sid * cpt, cpt)], slice_v)
        cnt = jnp.zeros((16,), jnp.int32)
        for j in range(cpt // 16):
            v = slice_v[pl.ds(j * 16, 16)]
            cnt = cnt + jnp.minimum(v, 1)
        cnt_v[...] = cnt
        pltpu.sync_copy(cnt_v, cnt_sh.at[pl.ds(sid * 16, 16)])
        plsc.subcore_barrier()

        @pl.when(sid == 0)
        def _finish():
            pltpu.sync_copy(cnt_sh, cnt_all_v)
            tot = jnp.zeros((16,), jnp.int32)
            for t in range(ns):
                tot = tot + cnt_all_v[pl.ds(t * 16, 16)]
            total = tot[0]
            for i in range(1, 16):
                total = total + tot[i]
            perp = total.astype(jnp.float32) * (1.0 / float(n))
            out_v[...] = lax.broadcast(perp, (16,))
            pltpu.sync_copy(out_v, perp_hbm)

    return sc_k(zc)


def kernel(z_e, grid):
    n, d = z_e.shape
    zc = jnp.transpose(z_e[:, :_CODE_DIMS])
    mixed2d, perp_vec = _sc_mixed_perp(zc, n)
    zq_st, loss2d = _quantize(z_e, grid)
    mixed = mixed2d.reshape(n)
    loss = loss2d.reshape(())
    perplexity = perp_vec[0]
    return zq_st, mixed, loss, perplexity


# speedup vs baseline: 1.0179x; 1.0179x over previous
import functools

import jax
import jax.numpy as jnp
from jax import lax
from jax.experimental import pallas as pl
from jax.experimental.pallas import tpu as pltpu
from jax.experimental.pallas import tpu_sc as plsc

_LEVELS = 8
_BETA = 0.25
_CODE_DIMS = 4


def _make_quant_body(nblk, n_rows, n_cols):
    inv = (1.0 + _BETA) / float(n_rows * n_cols)

    nlev = float(_LEVELS - 1)

    def body(z_ref, g_ref, zq_ref, loss_ref, acc_ref):
        z = z_ref[...]
        g0 = g_ref[0, 0]
        step = (g_ref[0, _LEVELS - 1] - g0) * (1.0 / (_LEVELS - 1))
        r = 1.0 / (jnp.exp(z + z) + 1.0)
        x = 1.0 - (r + r)
        best_f = jnp.floor((nlev + 0.5) - nlev * r)
        zq = g0 + best_f * step
        zq_ref[...] = x + (zq - x)
        diff = zq - x
        i = pl.program_id(0)

        @pl.when(i == 0)
        def _init():
            acc_ref[...] = jnp.zeros(acc_ref.shape, jnp.float32)

        acc_ref[...] += jnp.sum(diff * diff, axis=0, keepdims=True)

        @pl.when(i == nblk - 1)
        def _fin():
            loss_ref[...] = jnp.sum(acc_ref[...])[None, None] * inv

    return body


def _quantize(z_e, grid, nblk=8, interpret=False):
    n, d = z_e.shape
    k = grid.shape[0]
    r = n // nblk
    g2 = grid.reshape(1, k)
    return pl.pallas_call(
        _make_quant_body(nblk, n, d),
        grid=(nblk,),
        in_specs=[
            pl.BlockSpec((r, d), lambda i: (i, 0)),
            pl.BlockSpec((1, k), lambda i: (0, 0)),
        ],
        out_specs=[
            pl.BlockSpec((r, d), lambda i: (i, 0)),
            pl.BlockSpec((1, 1), lambda i: (0, 0)),
        ],
        out_shape=[
            jax.ShapeDtypeStruct((n, d), jnp.float32),
            jax.ShapeDtypeStruct((1, 1), jnp.float32),
        ],
        scratch_shapes=[pltpu.VMEM((1, d), jnp.float32)],
        interpret=interpret,
    )(z_e, g2)


def _sc_mixed_perp(zc, n):
    ns = 16
    rpt_n = n // ns
    mrows = rpt_n // 128
    codes = _LEVELS ** _CODE_DIMS
    cpt = codes // ns
    mesh = plsc.VectorSubcoreMesh(
        core_axis_name="c", subcore_axis_name="s", num_cores=1)

    @functools.partial(
        pl.kernel,
        out_type=[
            jax.ShapeDtypeStruct((n // 128, 128), jnp.int32),
            jax.ShapeDtypeStruct((16,), jnp.float32),
        ],
        mesh=mesh,
        compiler_params=pltpu.CompilerParams(use_tc_tiling_on_sc=False),
        scratch_types=[
            pltpu.VMEM((rpt_n,), jnp.float32),
            pltpu.VMEM((rpt_n,), jnp.float32),
            pltpu.VMEM((rpt_n,), jnp.float32),
            pltpu.VMEM((rpt_n,), jnp.float32),
            pltpu.VMEM((mrows, 128), jnp.int32),
            pltpu.VMEM((128,), jnp.int32),
            pltpu.VMEM((cpt,), jnp.int32),
            pltpu.VMEM((16,), jnp.int32),
            pltpu.VMEM((ns * 16,), jnp.int32),
            pltpu.VMEM((16,), jnp.float32),
            pltpu.VMEM_SHARED((codes,), jnp.int32),
            pltpu.VMEM_SHARED((ns * 16,), jnp.int32),
        ],
    )
    def sc_k(zc_hbm, mixed_hbm, perp_hbm, cb0, cb1, cb2, cb3,
             m2_v, ones_v, slice_v, cnt_v, cnt_all_v, out_v,
             counts_sh, cnt_sh):
        sid = lax.axis_index("s")
        base = sid * rpt_n
        pltpu.sync_copy(zc_hbm.at[0, pl.ds(base, rpt_n)], cb0)
        pltpu.sync_copy(zc_hbm.at[1, pl.ds(base, rpt_n)], cb1)
        pltpu.sync_copy(zc_hbm.at[2, pl.ds(base, rpt_n)], cb2)
        pltpu.sync_copy(zc_hbm.at[3, pl.ds(base, rpt_n)], cb3)
        one16 = jnp.ones((16,), jnp.int32)
        zero16 = jnp.zeros((16,), jnp.int32)
        for j in range(128 // 16):
            ones_v[pl.ds(j * 16, 16)] = one16
        for j in range(cpt // 16):
            slice_v[pl.ds(j * 16, 16)] = zero16
        pltpu.sync_copy(slice_v, counts_sh.at[pl.ds(sid * cpt, cpt)])
        cbs = (cb0, cb1, cb2, cb3)

        nlev = float(_LEVELS - 1)

        def quant_row(rr, _):
            for gg in range(8):
                off = (rr * 8 + gg) * 16
                m = jnp.zeros((16,), jnp.int32)
                for k in range(_CODE_DIMS):
                    v = cbs[k][pl.ds(off, 16)]
                    t = jnp.exp(v + v)
                    idx_k = (nlev + 0.5) - nlev / (t + 1.0)
                    m = m + idx_k.astype(jnp.int32) * (_LEVELS ** k)
                m2_v[rr, pl.ds(gg * 16, 16)] = m
            return 0

        lax.fori_loop(0, mrows, quant_row, 0)
        pltpu.sync_copy(m2_v, mixed_hbm.at[pl.ds(sid * mrows, mrows)])
        plsc.subcore_barrier()
        for j in range(mrows):
            pltpu.sync_copy(ones_v, counts_sh.at[m2_v.at[j]], add=True)
        plsc.subcore_barrier()
        pltpu.sync_copy(counts_sh.at[pl.ds(sid * cpt, cpt)], slice_v)
        cnt = jnp.zeros((16,), jnp.int32)
        for j in range(cpt // 16):
            v = slice_v[pl.ds(j * 16, 16)]
            cnt = cnt + jnp.minimum(v, 1)
        cnt_v[...] = cnt
        pltpu.sync_copy(cnt_v, cnt_sh.at[pl.ds(sid * 16, 16)])
        plsc.subcore_barrier()

        @pl.when(sid == 0)
        def _finish():
            pltpu.sync_copy(cnt_sh, cnt_all_v)
            tot = jnp.zeros((16,), jnp.int32)
            for t in range(ns):
                tot = tot + cnt_all_v[pl.ds(t * 16, 16)]
            total = tot[0]
            for i in range(1, 16):
                total = total + tot[i]
            perp = total.astype(jnp.float32) * (1.0 / float(n))
            out_v[...] = lax.broadcast(perp, (16,))
            pltpu.sync_copy(out_v, perp_hbm)

    return sc_k(zc)


def kernel(z_e, grid):
    n, d = z_e.shape
    zc = jnp.transpose(z_e[:, :_CODE_DIMS])
    mixed2d, perp_vec = _sc_mixed_perp(zc, n)
    zq_st, loss2d = _quantize(z_e, grid)
    mixed = mixed2d.reshape(n)
    loss = loss2d.reshape(())
    perplexity = perp_vec[0]
    return zq_st, mixed, loss, perplexity
